# Initial kernel scaffold; baseline (speedup 1.0000x reference)
import functools

import jax
import jax.numpy as jnp
from jax import lax
from jax.experimental import pallas as pl
from jax.experimental.pallas import tpu as pltpu
from jax.experimental.pallas import tpu_sc as plsc

N_NODES = 100000
N_EDGES = 3200000
NG = 10

NUM_CORES = 2
NUM_SUBCORES = 16
N_WORKERS = NUM_CORES * NUM_SUBCORES
PER_WORKER = N_EDGES // N_WORKERS
CHUNK = 2000
GROUPS = CHUNK // 16
N_CHUNKS = PER_WORKER // CHUNK

_MAGIC = jnp.int32(0x5F3759DF)


def _edge_body(src_hbm, dst_hbm, pos4_hbm, prm_hbm, out_hbm,
               srcv, dstv, pi, pj, outv, prmv, sem_i, sem_j):
    c = lax.axis_index("c")
    s = lax.axis_index("s")
    wid = s * NUM_CORES + c

    pltpu.sync_copy(prm_hbm, prmv)
    shifts = [prmv[k] for k in range(NG)]
    coefs = [prmv[NG + k] for k in range(NG)]
    iota = lax.iota(jnp.int32, 16)
    col_i32 = [jnp.full((16,), k, jnp.int32) for k in range(max(4, NG))]

    def chunk_body(ci, carry):
        base = wid * PER_WORKER + ci * CHUNK
        pltpu.sync_copy(src_hbm.at[pl.ds(base, CHUNK)], srcv)
        pltpu.sync_copy(dst_hbm.at[pl.ds(base, CHUNK)], dstv)
        cp_i = pltpu.async_copy(pos4_hbm.at[srcv], pi, sem_i)
        cp_j = pltpu.async_copy(pos4_hbm.at[dstv], pj, sem_j)
        cp_i.wait()
        cp_j.wait()

        def grp(g, carry2):
            row = g * 16 + iota
            xi = plsc.load_gather(pi, [row, col_i32[0]])
            yi = plsc.load_gather(pi, [row, col_i32[1]])
            zi = plsc.load_gather(pi, [row, col_i32[2]])
            xj = plsc.load_gather(pj, [row, col_i32[0]])
            yj = plsc.load_gather(pj, [row, col_i32[1]])
            zj = plsc.load_gather(pj, [row, col_i32[2]])
            dx = xi - xj
            dy = yi - yj
            dz = zi - zj
            d2 = jnp.maximum(dx * dx + dy * dy + dz * dz,
                             jnp.float32(1e-30))
            bits = lax.bitcast_convert_type(d2, jnp.int32)
            r = lax.bitcast_convert_type(
                _MAGIC - lax.shift_right_logical(bits, 1), jnp.float32)
            half = jnp.float32(0.5) * d2
            for _ in range(3):
                r = r * (jnp.float32(1.5) - half * r * r)
            length = d2 * r
            for k in range(NG):
                t = length - shifts[k]
                o = jnp.exp(t * t * coefs[k])
                plsc.store_scatter(outv, [row, col_i32[k]], o)
            return carry2

        lax.fori_loop(0, GROUPS, grp, 0)
        pltpu.sync_copy(outv, out_hbm.at[pl.ds(base, CHUNK)])
        return carry

    lax.fori_loop(0, N_CHUNKS, chunk_body, 0)


@jax.jit
def _gbf_sc(src, dst, pos4, prm):
    mesh = plsc.VectorSubcoreMesh(core_axis_name="c", subcore_axis_name="s")
    fn = pl.kernel(
        _edge_body,
        out_type=jax.ShapeDtypeStruct((N_EDGES, NG), jnp.float32),
        mesh=mesh,
        scratch_types=[
            pltpu.VMEM((CHUNK,), jnp.int32),
            pltpu.VMEM((CHUNK,), jnp.int32),
            pltpu.VMEM((CHUNK, 4), jnp.float32),
            pltpu.VMEM((CHUNK, 4), jnp.float32),
            pltpu.VMEM((CHUNK, NG), jnp.float32),
            pltpu.VMEM((2 * NG, 16), jnp.float32),
            pltpu.SemaphoreType.DMA,
            pltpu.SemaphoreType.DMA,
        ],
    )
    return fn(src, dst, pos4, prm)


def kernel(pos, edge_index, shift, scale):
    src = edge_index[0]
    dst = edge_index[1]
    pos4 = jnp.pad(pos, ((0, 0), (0, 1)))
    coef = -1.0 / (2.0 * scale * scale)
    prm = jnp.concatenate(
        [jnp.broadcast_to(shift[:, None], (NG, 16)),
         jnp.broadcast_to(coef[:, None], (NG, 16))], axis=0)
    return _gbf_sc(src, dst, pos4, prm)


# speedup vs baseline: 8.3365x; 8.3365x over previous
import jax
import jax.numpy as jnp
from jax import lax
from jax.experimental import pallas as pl
from jax.experimental.pallas import tpu as pltpu
from jax.experimental.pallas import tpu_sc as plsc

N_NODES = 100000
N_EDGES = 3200000
NG = 10

NUM_CORES = 2
NUM_SUBCORES = 16
N_WORKERS = NUM_CORES * NUM_SUBCORES
PER_WORKER = N_EDGES // N_WORKERS
CHUNK = 2000
GROUPS = CHUNK // 16
N_CHUNKS = PER_WORKER // CHUNK

_MAGIC = 0x5F3759DF


def _edge_body(src_hbm, dst_hbm, px_hbm, py_hbm, pz_hbm, prm_hbm, out_hbm,
               srcv, dstv, pix, piy, piz, pjx, pjy, pjz, outf, prmv,
               sem_a, sem_b):
    c = lax.axis_index("c")
    s = lax.axis_index("s")
    wid = s * NUM_CORES + c

    pltpu.sync_copy(prm_hbm, prmv)
    shifts = [prmv[k] for k in range(NG)]
    coefs = [prmv[NG + k] for k in range(NG)]
    iota = lax.iota(jnp.int32, 16)

    def chunk_body(ci, carry):
        base = wid * PER_WORKER + ci * CHUNK
        pltpu.sync_copy(src_hbm.at[pl.ds(base, CHUNK)], srcv)
        pltpu.sync_copy(dst_hbm.at[pl.ds(base, CHUNK)], dstv)
        cps = [
            pltpu.async_copy(px_hbm.at[srcv], pix, sem_a),
            pltpu.async_copy(py_hbm.at[srcv], piy, sem_a),
            pltpu.async_copy(pz_hbm.at[srcv], piz, sem_a),
            pltpu.async_copy(px_hbm.at[dstv], pjx, sem_b),
            pltpu.async_copy(py_hbm.at[dstv], pjy, sem_b),
            pltpu.async_copy(pz_hbm.at[dstv], pjz, sem_b),
        ]
        for cp in cps:
            cp.wait()

        def grp(g, carry2):
            sl = pl.ds(g * 16, 16)
            dx = pix[sl] - pjx[sl]
            dy = piy[sl] - pjy[sl]
            dz = piz[sl] - pjz[sl]
            d2 = jnp.maximum(dx * dx + dy * dy + dz * dz,
                             jnp.float32(1e-30))
            bits = lax.bitcast_convert_type(d2, jnp.int32)
            r = lax.bitcast_convert_type(
                jnp.int32(_MAGIC) - lax.shift_right_logical(bits, 1),
                jnp.float32)
            half = jnp.float32(0.5) * d2
            for _ in range(3):
                r = r * (jnp.float32(1.5) - half * r * r)
            length = d2 * r
            orow = (g * 16 + iota) * NG
            for k in range(NG):
                t = length - shifts[k]
                o = jnp.exp(t * t * coefs[k])
                plsc.store_scatter(outf, [orow + k], o)
            return carry2

        lax.fori_loop(0, GROUPS, grp, 0)
        pltpu.sync_copy(outf, out_hbm.at[pl.ds(base * NG, CHUNK * NG)])
        return carry

    lax.fori_loop(0, N_CHUNKS, chunk_body, 0)


@jax.jit
def _gbf_sc(src, dst, px, py, pz, prm):
    mesh = plsc.VectorSubcoreMesh(core_axis_name="c", subcore_axis_name="s")
    fn = pl.kernel(
        _edge_body,
        out_type=jax.ShapeDtypeStruct((N_EDGES * NG,), jnp.float32),
        mesh=mesh,
        compiler_params=pltpu.CompilerParams(needs_layout_passes=False),
        scratch_types=[
            pltpu.VMEM((CHUNK,), jnp.int32),
            pltpu.VMEM((CHUNK,), jnp.int32),
            pltpu.VMEM((CHUNK,), jnp.float32),
            pltpu.VMEM((CHUNK,), jnp.float32),
            pltpu.VMEM((CHUNK,), jnp.float32),
            pltpu.VMEM((CHUNK,), jnp.float32),
            pltpu.VMEM((CHUNK,), jnp.float32),
            pltpu.VMEM((CHUNK,), jnp.float32),
            pltpu.VMEM((CHUNK * NG,), jnp.float32),
            pltpu.VMEM((2 * NG, 16), jnp.float32),
            pltpu.SemaphoreType.DMA,
            pltpu.SemaphoreType.DMA,
        ],
    )
    return fn(src, dst, px, py, pz, prm)


def kernel(pos, edge_index, shift, scale):
    src = edge_index[0]
    dst = edge_index[1]
    px = pos[:, 0]
    py = pos[:, 1]
    pz = pos[:, 2]
    coef = -1.0 / (2.0 * scale * scale)
    prm = jnp.concatenate(
        [jnp.broadcast_to(shift[:, None], (NG, 16)),
         jnp.broadcast_to(coef[:, None], (NG, 16))], axis=0)
    out = _gbf_sc(src, dst, px, py, pz, prm)
    return out.reshape(N_EDGES, NG)
